# Initial kernel scaffold; baseline (speedup 1.0000x reference)
import jax
import jax.numpy as jnp
from jax.experimental import pallas as pl


def kernel(logits, labels):
    raise NotImplementedError("write your pallas kernel here")


# speedup vs baseline: 1.7004x; 1.7004x over previous
import jax
import jax.numpy as jnp
from jax.experimental import pallas as pl
from jax.experimental.pallas import tpu as pltpu

N_BINS_K = 15
ROW_BLOCK = 8000


def _ece_body(nblocks, total_rows):
    def body(logits_ref, labels_ref, uppers_ref, out_ref, hist_ref):
        i = pl.program_id(0)

        @pl.when(i == 0)
        def _init():
            hist_ref[...] = jnp.zeros_like(hist_ref)

        x = logits_ref[...]
        b, c = x.shape
        m = jnp.max(x, axis=1, keepdims=True)
        s = jnp.sum(jnp.exp(x - m), axis=1)
        conf = 1.0 / s
        pred = jnp.argmax(x, axis=1).astype(jnp.int32)
        labels = labels_ref[0, 0, :]
        accv = (pred == labels).astype(jnp.float32)

        uppers = uppers_ref[0, 0, :]
        bin_idx = jnp.sum(
            (uppers[None, :] < conf[:, None]).astype(jnp.int32), axis=1)
        bin_idx = jnp.minimum(bin_idx, N_BINS_K - 1)

        row = i * b + jax.lax.broadcasted_iota(jnp.int32, (b, 1), 0)
        valid = (row < total_rows).astype(jnp.float32)

        ponehot = (jax.lax.broadcasted_iota(jnp.int32, (b, c), 1)
                   == pred[:, None]).astype(jnp.float32) * valid
        binoh = (jax.lax.broadcasted_iota(jnp.int32, (b, N_BINS_K), 1)
                 == bin_idx[:, None]).astype(jnp.float32)
        vals = jnp.concatenate(
            [binoh, binoh * conf[:, None], binoh * accv[:, None]], axis=1)

        hist_ref[...] += jax.lax.dot_general(
            ponehot, vals, (((0,), (0,)), ((), ())),
            preferred_element_type=jnp.float32,
            precision=jax.lax.Precision.HIGHEST)

        @pl.when(i == nblocks - 1)
        def _fin():
            h = hist_ref[...]
            count = h[:, :N_BINS_K]
            conf_sum = h[:, N_BINS_K:2 * N_BINS_K]
            acc_sum = h[:, 2 * N_BINS_K:3 * N_BINS_K]
            class_count = jnp.sum(count, axis=1, keepdims=True)
            safe = jnp.maximum(count, 1.0)
            prop = count / jnp.maximum(class_count, 1.0)
            gap = jnp.where(count > 0.0,
                            jnp.abs(conf_sum / safe - acc_sum / safe) * prop,
                            0.0)
            out_ref[...] = jnp.sum(gap, axis=1)[None, :]

    return body


def kernel(logits, labels):
    n, c = logits.shape
    labels = labels.astype(jnp.int32)

    b = ROW_BLOCK
    nblocks = -(-n // b)
    npad = nblocks * b
    if npad != n:
        logits = jnp.pad(logits, ((0, npad - n), (0, 0)))
        labels = jnp.pad(labels, (0, npad - n))
    labels3 = labels.reshape(nblocks, 1, b)
    uppers = jnp.linspace(0.0, 1.0, N_BINS_K + 1)[1:].astype(
        jnp.float32).reshape(1, 1, N_BINS_K)

    out = pl.pallas_call(
        _ece_body(nblocks, n),
        grid=(nblocks,),
        in_specs=[
            pl.BlockSpec((b, c), lambda i: (i, 0)),
            pl.BlockSpec((1, 1, b), lambda i: (i, 0, 0)),
            pl.BlockSpec((1, 1, N_BINS_K), lambda i: (0, 0, 0)),
        ],
        out_specs=pl.BlockSpec((1, c), lambda i: (0, 0)),
        out_shape=jax.ShapeDtypeStruct((1, c), jnp.float32),
        scratch_shapes=[pltpu.VMEM((c, 3 * N_BINS_K), jnp.float32)],
    )(logits, labels3, uppers)
    return out.reshape(c)


# speedup vs baseline: 4.7342x; 2.7841x over previous
import jax
import jax.numpy as jnp
from jax.experimental import pallas as pl
from jax.experimental.pallas import tpu as pltpu

N_BINS_K = 15
ROW_BLOCK = 8000


def _ece_body(nblocks, total_rows):
    def body(logits_ref, labels_ref, uppers_ref, out_ref, hist_ref):
        i = pl.program_id(0)

        @pl.when(i == 0)
        def _init():
            hist_ref[...] = jnp.zeros_like(hist_ref)

        x = logits_ref[...]
        b, c = x.shape
        xt = x.T
        m = jnp.max(xt, axis=0, keepdims=True)
        s = jnp.sum(jnp.exp(xt - m), axis=0, keepdims=True)
        conf = 1.0 / s
        cls_iota = jax.lax.broadcasted_iota(jnp.int32, (c, b), 0)
        cand = jnp.where(xt == m, cls_iota, c)
        pred = jnp.min(cand, axis=0, keepdims=True)
        labels = labels_ref[...][:, 0, :]
        accv = (pred == labels).astype(jnp.float32)

        uppers = uppers_ref[...][0]
        lowers = uppers_ref[...][1]
        binoh = jnp.logical_and(lowers < conf, conf <= uppers)
        if nblocks * b != total_rows:
            col = i * b + jax.lax.broadcasted_iota(jnp.int32, (1, b), 1)
            binoh = jnp.logical_and(binoh, col < total_rows)
        conf_hi = conf.astype(jnp.bfloat16).astype(jnp.float32)
        conf_lo = conf - conf_hi
        valsT = jnp.concatenate(
            [jnp.where(binoh, 1.0, 0.0),
             jnp.where(binoh, accv, 0.0),
             jnp.where(binoh, conf_hi, 0.0),
             jnp.where(binoh, conf_lo, 0.0)], axis=0
        ).astype(jnp.bfloat16)

        ponehot = jnp.where(cand == pred, 1.0, 0.0).astype(jnp.bfloat16)

        hist_ref[...] += jax.lax.dot_general(
            ponehot, valsT, (((1,), (1,)), ((), ())),
            preferred_element_type=jnp.float32)

        @pl.when(i == nblocks - 1)
        def _fin():
            h = hist_ref[...]
            count = h[:, :N_BINS_K]
            acc_sum = h[:, N_BINS_K:2 * N_BINS_K]
            conf_sum = (h[:, 2 * N_BINS_K:3 * N_BINS_K]
                        + h[:, 3 * N_BINS_K:4 * N_BINS_K])
            class_count = jnp.sum(count, axis=1, keepdims=True)
            safe = jnp.maximum(count, 1.0)
            prop = count / jnp.maximum(class_count, 1.0)
            gap = jnp.where(count > 0.0,
                            jnp.abs(conf_sum / safe - acc_sum / safe) * prop,
                            0.0)
            out_ref[...] = jnp.sum(gap, axis=1)[None, :]

    return body


def kernel(logits, labels):
    n, c = logits.shape
    labels = labels.astype(jnp.int32)

    b = ROW_BLOCK
    nblocks = -(-n // b)
    npad = nblocks * b
    if npad != n:
        logits = jnp.pad(logits, ((0, npad - n), (0, 0)))
        labels = jnp.pad(labels, (0, npad - n))
    labels3 = labels.reshape(nblocks, 1, b)
    boundaries = jnp.linspace(0.0, 1.0, N_BINS_K + 1).astype(jnp.float32)
    uppers = boundaries[1:]
    lowers = jnp.concatenate(
        [jnp.full((1,), -jnp.inf, jnp.float32), boundaries[1:N_BINS_K]])
    uppers = uppers.at[N_BINS_K - 1].set(jnp.inf)
    bnds = jnp.stack([uppers, lowers]).reshape(2, N_BINS_K, 1)

    out = pl.pallas_call(
        _ece_body(nblocks, n),
        grid=(nblocks,),
        in_specs=[
            pl.BlockSpec((b, c), lambda i: (i, 0)),
            pl.BlockSpec((1, 1, b), lambda i: (i, 0, 0)),
            pl.BlockSpec((2, N_BINS_K, 1), lambda i: (0, 0, 0)),
        ],
        out_specs=pl.BlockSpec((1, c), lambda i: (0, 0)),
        out_shape=jax.ShapeDtypeStruct((1, c), jnp.float32),
        scratch_shapes=[pltpu.VMEM((c, 4 * N_BINS_K), jnp.float32)],
    )(logits, labels3, bnds)
    return out.reshape(c)
